# baseline (speedup 1.0000x reference)
import jax
import jax.numpy as jnp
from jax.experimental import pallas as pl
from jax.experimental.pallas import tpu as pltpu

B, C, H, W = 64, 192, 56, 56
NPIX = B * H * W
PBLK = 7168
NGRID = NPIX // PBLK


def _permute_block(x_ref, m_ref, o_ref):
    xb = x_ref[...]
    hi = xb.astype(jnp.bfloat16).astype(jnp.float32)
    r = xb - hi
    mid = r.astype(jnp.bfloat16).astype(jnp.float32)
    lo = r - mid
    mm = m_ref[...]
    o_ref[...] = (jnp.dot(hi, mm, preferred_element_type=jnp.float32)
                  + jnp.dot(mid, mm, preferred_element_type=jnp.float32)
                  + jnp.dot(lo, mm, preferred_element_type=jnp.float32))


def _lane_permute(x2, m):
    return pl.pallas_call(
        _permute_block,
        grid=(NGRID,),
        in_specs=[
            pl.BlockSpec((PBLK, C), lambda i: (i, 0)),
            pl.BlockSpec((C, C), lambda i: (0, 0)),
        ],
        out_specs=pl.BlockSpec((PBLK, C), lambda i: (i, 0)),
        out_shape=jax.ShapeDtypeStruct((NPIX, C), jnp.float32),
        compiler_params=pltpu.CompilerParams(
            dimension_semantics=("arbitrary",),
        ),
    )(x2, m)


@jax.jit
def kernel(x, permutation):
    x2 = x.transpose(0, 2, 3, 1).reshape(NPIX, C)
    m = (permutation[None, :] == jnp.arange(C, dtype=permutation.dtype)[:, None]
         ).astype(jnp.float32)
    out2 = _lane_permute(x2, m)
    z = out2.reshape(B, H, W, C).transpose(0, 3, 1, 2)
    ldj = jnp.zeros((B,), dtype=x.dtype)
    return (z, ldj)
